# baseline (speedup 1.0000x reference)
import functools

import jax
import jax.numpy as jnp
from jax import lax
from jax.experimental import pallas as pl
from jax.experimental.pallas import tpu as pltpu
from jax.experimental.pallas import tpu_sc as plsc

_NC, _NS = 2, 16
_NW = _NC * _NS
_SLICES = [160000, 160000]


def _ln(o, g, b, eps=1e-5):
    mu = jnp.mean(o, axis=-1, keepdims=True)
    d = o - mu
    var = jnp.mean(d * d, axis=-1, keepdims=True)
    return d * lax.rsqrt(var + eps) * g + b


def _silu(x):
    return x * jax.nn.sigmoid(x)


def _grid_body(x_ref, W1_ref, b1_ref, W2_ref, b2_ref, g_ref, bn_ref, out_ref):
    x = x_ref[...]
    h = _silu(jnp.dot(x, W1_ref[...], preferred_element_type=jnp.float32)
              + b1_ref[...])
    o = jnp.dot(h, W2_ref[...], preferred_element_type=jnp.float32) + b2_ref[...]
    out_ref[...] = x + _ln(o, g_ref[...], bn_ref[...])


def _proj_body(x_ref, W_ref, out_ref):
    out_ref[...] = jnp.dot(x_ref[...], W_ref[...],
                           preferred_element_type=jnp.float32)


def _edge_body(e_ref, gsum_ref, W1_ref, b1_ref, W2_ref, b2_ref, g_ref, bn_ref,
               out_ref):
    pre = (jnp.dot(e_ref[...], W1_ref[...], preferred_element_type=jnp.float32)
           + gsum_ref[...] + b1_ref[...])
    h = _silu(pre)
    o = jnp.dot(h, W2_ref[...], preferred_element_type=jnp.float32) + b2_ref[...]
    out_ref[...] = _ln(o, g_ref[...], bn_ref[...])


def _mesh_body(*refs):
    np_ = len(refs) - 9
    p_refs, (m_ref, Wa_ref, Wm_ref, b1_ref, W2_ref, b2_ref, g_ref, bn_ref,
             out_ref) = refs[:np_], refs[np_:]
    agg = p_refs[0][0] + p_refs[0][1]
    for p in p_refs[1:]:
        agg = agg + p[0] + p[1]
    m = m_ref[...]
    pre = (jnp.dot(agg, Wa_ref[...], preferred_element_type=jnp.float32)
           + jnp.dot(m, Wm_ref[...], preferred_element_type=jnp.float32)
           + b1_ref[...])
    h = _silu(pre)
    o = jnp.dot(h, W2_ref[...], preferred_element_type=jnp.float32) + b2_ref[...]
    out_ref[...] = m + _ln(o, g_ref[...], bn_ref[...])


def _full(shape):
    n = len(shape)
    return pl.BlockSpec(shape, lambda i: (0,) * n)


def _make_gather_add(E_sl, e_off, D):
    per_w = E_sl // _NW
    CH = 40
    NB = 5
    n_grp = per_w // (CH * NB)
    assert CH * NB * n_grp == per_w
    mesh = plsc.VectorSubcoreMesh(core_axis_name="c", subcore_axis_name="s",
                                  num_cores=_NC, num_subcores=_NS)

    @functools.partial(
        pl.kernel,
        out_type=jax.ShapeDtypeStruct((E_sl, D), jnp.float32),
        mesh=mesh,
        scratch_types=[
            pltpu.VMEM((per_w,), jnp.int32),
            pltpu.VMEM((per_w,), jnp.int32),
            pltpu.VMEM((NB, CH, D), jnp.float32),
            pltpu.VMEM((NB, CH, D), jnp.float32),
        ] + [pltpu.SemaphoreType.DMA] * (3 * NB),
    )
    def gather_add(ge_hbm, me_hbm, src_hbm, dst_hbm, out_hbm,
                   sidx, didx, ra, rb, *sems):
        ga, gb, ws = sems[:NB], sems[NB:2 * NB], sems[2 * NB:]
        wid = lax.axis_index("c") * _NS + lax.axis_index("s")
        base = wid * per_w
        pltpu.sync_copy(src_hbm.at[pl.ds(e_off + base, per_w)], sidx)
        pltpu.sync_copy(dst_hbm.at[pl.ds(e_off + base, per_w)], didx)

        def group(g, carry):
            k0 = g * (CH * NB)
            descs = []
            for b in range(NB):
                off = k0 + b * CH
                @pl.when(g > 0)
                def _():
                    pltpu.make_async_copy(
                        ra.at[b], out_hbm.at[pl.ds(base + off, CH)],
                        ws[b]).wait()
                da = pltpu.async_copy(
                    ge_hbm.at[sidx.at[pl.ds(off, CH)]], ra.at[b], ga[b])
                db = pltpu.async_copy(
                    me_hbm.at[didx.at[pl.ds(off, CH)]], rb.at[b], gb[b])
                descs.append((da, db))
            for b in range(NB):
                off = k0 + b * CH
                da, db = descs[b]
                da.wait()
                db.wait()

                def row(r, c2):
                    for j in range(D // 16):
                        sl = pl.ds(j * 16, 16)
                        ra[b, r, sl] = ra[b, r, sl] + rb[b, r, sl]
                    return c2

                lax.fori_loop(0, CH, row, 0)
                pltpu.async_copy(ra.at[b], out_hbm.at[pl.ds(base + off, CH)],
                                 ws[b])
            return carry

        lax.fori_loop(0, n_grp, group, 0)
        for b in range(NB):
            pltpu.make_async_copy(
                ra.at[b], out_hbm.at[pl.ds(base + b * CH, CH)], ws[b]).wait()

    return gather_add


def _make_scatter_sum(E_sl, e_off, Nm, D):
    per_w = E_sl // _NW
    CH = 40
    NB = 5
    n_grp = per_w // (CH * NB)
    assert CH * NB * n_grp == per_w
    n_blk = Nm // CH
    mesh = plsc.VectorSubcoreMesh(core_axis_name="c", subcore_axis_name="s",
                                  num_cores=_NC, num_subcores=_NS)

    @functools.partial(
        pl.kernel,
        out_type=jax.ShapeDtypeStruct((_NC * Nm, D), jnp.float32),
        mesh=mesh,
        scratch_types=[
            pltpu.VMEM((NB, CH), jnp.int32),
            pltpu.VMEM((NB, CH, D), jnp.float32),
            pltpu.VMEM_SHARED((Nm, D), jnp.float32),
        ] + [pltpu.SemaphoreType.DMA] * (3 * NB),
    )
    def scatter_sum(ef_hbm, dst_hbm, out_hbm, didx, rows, acc, *sems):
        ri, rr, ss = sems[:NB], sems[NB:2 * NB], sems[2 * NB:]
        c = lax.axis_index("c")
        s = lax.axis_index("s")
        base = (c * _NS + s) * per_w
        n_own = (n_blk - s + _NS - 1) // _NS

        def zrow(r, carry):
            for j in range(D // 16):
                rows[0, r, pl.ds(j * 16, 16)] = jnp.zeros((16,), jnp.float32)
            return carry

        lax.fori_loop(0, CH, zrow, 0)

        def zblk(m, carry):
            blk = s + m * _NS
            pltpu.sync_copy(rows.at[0], acc.at[pl.ds(blk * CH, CH)])
            return carry

        lax.fori_loop(0, n_own, zblk, 0)
        plsc.subcore_barrier()

        def group(g, carry):
            k0 = base + g * (CH * NB)
            descs = []
            for b in range(NB):
                off = k0 + b * CH
                @pl.when(g > 0)
                def _():
                    pltpu.make_async_copy(rows.at[b], acc.at[didx.at[b]],
                                          ss[b]).wait()
                di = pltpu.async_copy(dst_hbm.at[pl.ds(e_off + off, CH)],
                                      didx.at[b], ri[b])
                dr = pltpu.async_copy(ef_hbm.at[pl.ds(off, CH)],
                                      rows.at[b], rr[b])
                descs.append((di, dr))
            for b in range(NB):
                di, dr = descs[b]
                di.wait()
                dr.wait()
                pltpu.async_copy(rows.at[b], acc.at[didx.at[b]], ss[b],
                                 add=True)
            return carry

        lax.fori_loop(0, n_grp, group, 0)
        for b in range(NB):
            pltpu.make_async_copy(rows.at[b], acc.at[didx.at[b]],
                                  ss[b]).wait()
        plsc.subcore_barrier()

        def oblk(m, carry):
            blk = s + m * _NS
            pltpu.sync_copy(acc.at[pl.ds(blk * CH, CH)],
                            out_hbm.at[pl.ds(c * Nm + blk * CH, CH)])
            return carry

        lax.fori_loop(0, n_own, oblk, 0)

    return scatter_sum


def kernel(g2m_efeat, grid_nfeat, mesh_nfeat, src_idx, dst_idx,
           e_W1, e_b1, e_W2, e_b2, e_g, e_bn,
           s_W1, s_b1, s_W2, s_b2, s_g, s_bn,
           d_W1, d_b1, d_W2, d_b2, d_g, d_bn):
    E, D = g2m_efeat.shape
    Ng = grid_nfeat.shape[0]
    Nm = mesh_nfeat.shape[0]
    H = e_W1.shape[1]
    f32 = jnp.float32
    assert sum(_SLICES) == E

    W1e = e_W1[:D]
    W1g = e_W1[D:2 * D]
    W1d = e_W1[2 * D:]
    d_W1a = d_W1[:D]
    d_W1m = d_W1[D:]

    r2 = lambda v: v.reshape(1, -1)

    BG = 1000
    Ge = pl.pallas_call(
        _proj_body,
        grid=(Ng // BG,),
        in_specs=[pl.BlockSpec((BG, D), lambda i: (i, 0)), _full((D, H))],
        out_specs=pl.BlockSpec((BG, H), lambda i: (i, 0)),
        out_shape=jax.ShapeDtypeStruct((Ng, H), f32),
    )(grid_nfeat, W1g)

    BM = 1000
    Me = pl.pallas_call(
        _proj_body,
        grid=(Nm // BM,),
        in_specs=[pl.BlockSpec((BM, D), lambda i: (i, 0)), _full((D, H))],
        out_specs=pl.BlockSpec((BM, H), lambda i: (i, 0)),
        out_shape=jax.ShapeDtypeStruct((Nm, H), f32),
    )(mesh_nfeat, W1d)

    offs = [sum(_SLICES[:i]) for i in range(len(_SLICES))]
    gathered = [
        _make_gather_add(E_sl, off, H)(Ge, Me, src_idx, dst_idx)
        for E_sl, off in zip(_SLICES, offs)
    ]

    BE = 1600
    efeat = []
    for sl, (E_sl, off) in enumerate(zip(_SLICES, offs)):
        nblk = E_sl // BE
        off_blk = off // BE
        efeat.append(pl.pallas_call(
            _edge_body,
            grid=(nblk,),
            in_specs=[
                pl.BlockSpec((BE, D), lambda i, o=off_blk: (i + o, 0)),
                pl.BlockSpec((BE, H), lambda i: (i, 0)),
                _full((D, H)), _full((1, H)), _full((H, D)), _full((1, D)),
                _full((1, D)), _full((1, D)),
            ],
            out_specs=pl.BlockSpec((BE, D), lambda i: (i, 0)),
            out_shape=jax.ShapeDtypeStruct((E_sl, D), f32),
        )(g2m_efeat, gathered[sl], W1e, r2(e_b1), e_W2, r2(e_b2), r2(e_g),
          r2(e_bn)))

    partials = [
        _make_scatter_sum(E_sl, off, Nm, D)(efeat[sl], dst_idx)
        .reshape(_NC, Nm, D)
        for sl, (E_sl, off) in enumerate(zip(_SLICES, offs))
    ]

    grid_out = pl.pallas_call(
        _grid_body,
        grid=(Ng // BG,),
        in_specs=[
            pl.BlockSpec((BG, D), lambda i: (i, 0)),
            _full((D, H)), _full((1, H)), _full((H, D)), _full((1, D)),
            _full((1, D)), _full((1, D)),
        ],
        out_specs=pl.BlockSpec((BG, D), lambda i: (i, 0)),
        out_shape=jax.ShapeDtypeStruct((Ng, D), f32),
    )(grid_nfeat, s_W1, r2(s_b1), s_W2, r2(s_b2), r2(s_g), r2(s_bn))

    mesh_out = pl.pallas_call(
        _mesh_body,
        grid=(Nm // BM,),
        in_specs=(
            [pl.BlockSpec((_NC, BM, D), lambda i: (0, i, 0))
             for _ in partials] +
            [pl.BlockSpec((BM, D), lambda i: (i, 0)),
             _full((D, H)), _full((D, H)), _full((1, H)), _full((H, D)),
             _full((1, D)), _full((1, D)), _full((1, D))]
        ),
        out_specs=pl.BlockSpec((BM, D), lambda i: (i, 0)),
        out_shape=jax.ShapeDtypeStruct((Nm, D), f32),
    )(*partials, mesh_nfeat, d_W1a, d_W1m,
      r2(d_b1), d_W2, r2(d_b2), r2(d_g), r2(d_bn))

    return (grid_out, mesh_out)


# speedup vs baseline: 1.0379x; 1.0379x over previous
import functools

import jax
import jax.numpy as jnp
from jax import lax
from jax.experimental import pallas as pl
from jax.experimental.pallas import tpu as pltpu
from jax.experimental.pallas import tpu_sc as plsc

_NC, _NS = 2, 16
_NW = _NC * _NS
_SLICES = [160000, 160000]


def _ln(o, g, b, eps=1e-5):
    mu = jnp.mean(o, axis=-1, keepdims=True)
    d = o - mu
    var = jnp.mean(d * d, axis=-1, keepdims=True)
    return d * lax.rsqrt(var + eps) * g + b


def _silu(x):
    return x * jax.nn.sigmoid(x)


def _grid_body(x_ref, W1_ref, b1_ref, W2_ref, b2_ref, g_ref, bn_ref, out_ref):
    x = x_ref[...]
    h = _silu(jnp.dot(x, W1_ref[...], preferred_element_type=jnp.float32)
              + b1_ref[...])
    o = jnp.dot(h, W2_ref[...], preferred_element_type=jnp.float32) + b2_ref[...]
    out_ref[...] = x + _ln(o, g_ref[...], bn_ref[...])


def _proj_body(x_ref, W_ref, out_ref):
    out_ref[...] = jnp.dot(x_ref[...], W_ref[...],
                           preferred_element_type=jnp.float32)


def _edge_body(e_ref, gsum_ref, W1_ref, b1_ref, W2_ref, b2_ref, g_ref, bn_ref,
               out_ref):
    pre = (jnp.dot(e_ref[...], W1_ref[...], preferred_element_type=jnp.float32)
           + gsum_ref[...] + b1_ref[...])
    h = _silu(pre)
    o = jnp.dot(h, W2_ref[...], preferred_element_type=jnp.float32) + b2_ref[...]
    out_ref[...] = _ln(o, g_ref[...], bn_ref[...])


def _mesh_body(*refs):
    np_ = len(refs) - 9
    p_refs, (m_ref, Wa_ref, Wm_ref, b1_ref, W2_ref, b2_ref, g_ref, bn_ref,
             out_ref) = refs[:np_], refs[np_:]
    agg = p_refs[0][0] + p_refs[0][1]
    for p in p_refs[1:]:
        agg = agg + p[0] + p[1]
    m = m_ref[...]
    pre = (jnp.dot(agg, Wa_ref[...], preferred_element_type=jnp.float32)
           + jnp.dot(m, Wm_ref[...], preferred_element_type=jnp.float32)
           + b1_ref[...])
    h = _silu(pre)
    o = jnp.dot(h, W2_ref[...], preferred_element_type=jnp.float32) + b2_ref[...]
    out_ref[...] = m + _ln(o, g_ref[...], bn_ref[...])


def _full(shape):
    n = len(shape)
    return pl.BlockSpec(shape, lambda i: (0,) * n)


def _make_gather_add(E_sl, e_off, D):
    per_w = E_sl // _NW
    CH = 40
    NB = 5
    n_grp = per_w // (CH * NB)
    assert CH * NB * n_grp == per_w
    mesh = plsc.VectorSubcoreMesh(core_axis_name="c", subcore_axis_name="s",
                                  num_cores=_NC, num_subcores=_NS)

    @functools.partial(
        pl.kernel,
        out_type=jax.ShapeDtypeStruct((E_sl, D), jnp.float32),
        mesh=mesh,
        scratch_types=[
            pltpu.VMEM((per_w,), jnp.int32),
            pltpu.VMEM((per_w,), jnp.int32),
            pltpu.VMEM((NB, CH, D), jnp.float32),
            pltpu.VMEM((NB, CH, D), jnp.float32),
        ] + [pltpu.SemaphoreType.DMA] * (3 * NB),
    )
    def gather_add(ge_hbm, me_hbm, src_hbm, dst_hbm, out_hbm,
                   sidx, didx, ra, rb, *sems):
        ga, gb, ws = sems[:NB], sems[NB:2 * NB], sems[2 * NB:]
        wid = lax.axis_index("c") * _NS + lax.axis_index("s")
        base = wid * per_w
        pltpu.sync_copy(src_hbm.at[pl.ds(e_off + base, per_w)], sidx)
        pltpu.sync_copy(dst_hbm.at[pl.ds(e_off + base, per_w)], didx)

        def group(g, carry):
            k0 = g * (CH * NB)
            descs = []
            for b in range(NB):
                off = k0 + b * CH
                @pl.when(g > 0)
                def _():
                    pltpu.make_async_copy(
                        ra.at[b], out_hbm.at[pl.ds(base + off, CH)],
                        ws[b]).wait()
                da = pltpu.async_copy(
                    ge_hbm.at[sidx.at[pl.ds(off, CH)]], ra.at[b], ga[b])
                db = pltpu.async_copy(
                    me_hbm.at[didx.at[pl.ds(off, CH)]], rb.at[b], gb[b])
                descs.append((da, db))
            for b in range(NB):
                off = k0 + b * CH
                da, db = descs[b]
                da.wait()
                db.wait()

                def row(r, c2):
                    for j in range(D // 16):
                        sl = pl.ds(j * 16, 16)
                        ra[b, r, sl] = ra[b, r, sl] + rb[b, r, sl]
                    return c2

                lax.fori_loop(0, CH, row, 0)
                pltpu.async_copy(ra.at[b], out_hbm.at[pl.ds(base + off, CH)],
                                 ws[b])
            return carry

        lax.fori_loop(0, n_grp, group, 0)
        for b in range(NB):
            pltpu.make_async_copy(
                ra.at[b], out_hbm.at[pl.ds(base + b * CH, CH)], ws[b]).wait()

    return gather_add


def _make_scatter_sum(E_sl, e_off, Nm, D):
    per_w = E_sl // _NW
    CH = 40
    NB = 5
    n_grp = per_w // (CH * NB)
    assert CH * NB * n_grp == per_w
    n_blk = Nm // CH
    mesh = plsc.VectorSubcoreMesh(core_axis_name="c", subcore_axis_name="s",
                                  num_cores=_NC, num_subcores=_NS)

    @functools.partial(
        pl.kernel,
        out_type=jax.ShapeDtypeStruct((_NC * Nm, D), jnp.float32),
        mesh=mesh,
        scratch_types=[
            pltpu.VMEM((NB, CH), jnp.int32),
            pltpu.VMEM((NB, CH, D), jnp.float32),
            pltpu.VMEM_SHARED((Nm, D), jnp.float32),
        ] + [pltpu.SemaphoreType.DMA] * (3 * NB),
    )
    def scatter_sum(ef_hbm, dst_hbm, out_hbm, didx, rows, acc, *sems):
        ri, rr, ss = sems[:NB], sems[NB:2 * NB], sems[2 * NB:]
        c = lax.axis_index("c")
        s = lax.axis_index("s")
        base = (c * _NS + s) * per_w
        n_own = (n_blk - s + _NS - 1) // _NS

        def zrow(r, carry):
            for j in range(D // 16):
                rows[0, r, pl.ds(j * 16, 16)] = jnp.zeros((16,), jnp.float32)
            return carry

        lax.fori_loop(0, CH, zrow, 0)

        def zblk(m, carry):
            blk = s + m * _NS
            pltpu.sync_copy(rows.at[0], acc.at[pl.ds(blk * CH, CH)])
            return carry

        lax.fori_loop(0, n_own, zblk, 0)
        plsc.subcore_barrier()

        def group(g, carry):
            k0 = base + g * (CH * NB)
            descs = []
            for b in range(NB):
                off = k0 + b * CH
                @pl.when(g > 0)
                def _():
                    pltpu.make_async_copy(rows.at[b], acc.at[didx.at[b]],
                                          ss[b]).wait()
                di = pltpu.async_copy(dst_hbm.at[pl.ds(e_off + off, CH)],
                                      didx.at[b], ri[b])
                dr = pltpu.async_copy(ef_hbm.at[pl.ds(off, CH)],
                                      rows.at[b], rr[b])
                descs.append((di, dr))
            for b in range(NB):
                di, dr = descs[b]
                di.wait()
                dr.wait()
                pltpu.async_copy(rows.at[b], acc.at[didx.at[b]], ss[b],
                                 add=True)
            return carry

        lax.fori_loop(0, n_grp, group, 0)
        for b in range(NB):
            pltpu.make_async_copy(rows.at[b], acc.at[didx.at[b]],
                                  ss[b]).wait()
        plsc.subcore_barrier()

        def oblk(m, carry):
            blk = s + m * _NS
            pltpu.sync_copy(acc.at[pl.ds(blk * CH, CH)],
                            out_hbm.at[pl.ds(c * Nm + blk * CH, CH)])
            return carry

        lax.fori_loop(0, n_own, oblk, 0)

    return scatter_sum


def kernel(g2m_efeat, grid_nfeat, mesh_nfeat, src_idx, dst_idx,
           e_W1, e_b1, e_W2, e_b2, e_g, e_bn,
           s_W1, s_b1, s_W2, s_b2, s_g, s_bn,
           d_W1, d_b1, d_W2, d_b2, d_g, d_bn):
    E, D = g2m_efeat.shape
    Ng = grid_nfeat.shape[0]
    Nm = mesh_nfeat.shape[0]
    H = e_W1.shape[1]
    f32 = jnp.float32
    assert sum(_SLICES) == E

    W1e = e_W1[:D]
    W1g = e_W1[D:2 * D]
    W1d = e_W1[2 * D:]
    d_W1a = d_W1[:D]
    d_W1m = d_W1[D:]

    r2 = lambda v: v.reshape(1, -1)

    BG = 1000
    Ge = pl.pallas_call(
        _proj_body,
        grid=(Ng // BG,),
        in_specs=[pl.BlockSpec((BG, D), lambda i: (i, 0)), _full((D, H))],
        out_specs=pl.BlockSpec((BG, H), lambda i: (i, 0)),
        out_shape=jax.ShapeDtypeStruct((Ng, H), f32),
    )(grid_nfeat, W1g)

    BM = 1000
    Me = pl.pallas_call(
        _proj_body,
        grid=(Nm // BM,),
        in_specs=[pl.BlockSpec((BM, D), lambda i: (i, 0)), _full((D, H))],
        out_specs=pl.BlockSpec((BM, H), lambda i: (i, 0)),
        out_shape=jax.ShapeDtypeStruct((Nm, H), f32),
    )(mesh_nfeat, W1d)

    offs = [sum(_SLICES[:i]) for i in range(len(_SLICES))]
    gathered = [
        _make_gather_add(E_sl, off, H)(Ge, Me, src_idx, dst_idx)
        for E_sl, off in zip(_SLICES, offs)
    ]

    BE = 2000
    efeat = []
    for sl, (E_sl, off) in enumerate(zip(_SLICES, offs)):
        nblk = E_sl // BE
        off_blk = off // BE
        efeat.append(pl.pallas_call(
            _edge_body,
            grid=(nblk,),
            in_specs=[
                pl.BlockSpec((BE, D), lambda i, o=off_blk: (i + o, 0)),
                pl.BlockSpec((BE, H), lambda i: (i, 0)),
                _full((D, H)), _full((1, H)), _full((H, D)), _full((1, D)),
                _full((1, D)), _full((1, D)),
            ],
            out_specs=pl.BlockSpec((BE, D), lambda i: (i, 0)),
            out_shape=jax.ShapeDtypeStruct((E_sl, D), f32),
        )(g2m_efeat, gathered[sl], W1e, r2(e_b1), e_W2, r2(e_b2), r2(e_g),
          r2(e_bn)))

    partials = [
        _make_scatter_sum(E_sl, off, Nm, D)(efeat[sl], dst_idx)
        .reshape(_NC, Nm, D)
        for sl, (E_sl, off) in enumerate(zip(_SLICES, offs))
    ]

    grid_out = pl.pallas_call(
        _grid_body,
        grid=(Ng // BG,),
        in_specs=[
            pl.BlockSpec((BG, D), lambda i: (i, 0)),
            _full((D, H)), _full((1, H)), _full((H, D)), _full((1, D)),
            _full((1, D)), _full((1, D)),
        ],
        out_specs=pl.BlockSpec((BG, D), lambda i: (i, 0)),
        out_shape=jax.ShapeDtypeStruct((Ng, D), f32),
    )(grid_nfeat, s_W1, r2(s_b1), s_W2, r2(s_b2), r2(s_g), r2(s_bn))

    mesh_out = pl.pallas_call(
        _mesh_body,
        grid=(Nm // BM,),
        in_specs=(
            [pl.BlockSpec((_NC, BM, D), lambda i: (0, i, 0))
             for _ in partials] +
            [pl.BlockSpec((BM, D), lambda i: (i, 0)),
             _full((D, H)), _full((D, H)), _full((1, H)), _full((H, D)),
             _full((1, D)), _full((1, D)), _full((1, D))]
        ),
        out_specs=pl.BlockSpec((BM, D), lambda i: (i, 0)),
        out_shape=jax.ShapeDtypeStruct((Nm, D), f32),
    )(*partials, mesh_nfeat, d_W1a, d_W1m,
      r2(d_b1), d_W2, r2(d_b2), r2(d_g), r2(d_bn))

    return (grid_out, mesh_out)


# speedup vs baseline: 1.0908x; 1.0510x over previous
import functools

import jax
import jax.numpy as jnp
from jax import lax
from jax.experimental import pallas as pl
from jax.experimental.pallas import tpu as pltpu
from jax.experimental.pallas import tpu_sc as plsc

_NC, _NS = 2, 16
_NW = _NC * _NS
_SLICES = [160000, 160000]


def _ln(o, g, b, eps=1e-5):
    mu = jnp.mean(o, axis=-1, keepdims=True)
    d = o - mu
    var = jnp.mean(d * d, axis=-1, keepdims=True)
    return d * lax.rsqrt(var + eps) * g + b


def _silu(x):
    return x * jax.nn.sigmoid(x)


def _grid_body(x_ref, W1_ref, b1_ref, W2_ref, b2_ref, g_ref, bn_ref, out_ref):
    x = x_ref[...]
    h = _silu(jnp.dot(x, W1_ref[...], preferred_element_type=jnp.float32)
              + b1_ref[...])
    o = jnp.dot(h, W2_ref[...], preferred_element_type=jnp.float32) + b2_ref[...]
    out_ref[...] = x + _ln(o, g_ref[...], bn_ref[...])


def _proj_body(x_ref, W_ref, out_ref):
    out_ref[...] = jnp.dot(x_ref[...], W_ref[...],
                           preferred_element_type=jnp.float32)


def _edge_body(e_ref, gsum_ref, W1_ref, b1_ref, W2_ref, b2_ref, g_ref, bn_ref,
               out_ref):
    pre = (jnp.dot(e_ref[...], W1_ref[...], preferred_element_type=jnp.float32)
           + gsum_ref[...] + b1_ref[...])
    h = _silu(pre)
    o = jnp.dot(h, W2_ref[...], preferred_element_type=jnp.float32) + b2_ref[...]
    out_ref[...] = _ln(o, g_ref[...], bn_ref[...])


def _mesh_body(*refs):
    np_ = len(refs) - 9
    p_refs, (m_ref, Wa_ref, Wm_ref, b1_ref, W2_ref, b2_ref, g_ref, bn_ref,
             out_ref) = refs[:np_], refs[np_:]
    agg = p_refs[0][0] + p_refs[0][1]
    for p in p_refs[1:]:
        agg = agg + p[0] + p[1]
    m = m_ref[...]
    pre = (jnp.dot(agg, Wa_ref[...], preferred_element_type=jnp.float32)
           + jnp.dot(m, Wm_ref[...], preferred_element_type=jnp.float32)
           + b1_ref[...])
    h = _silu(pre)
    o = jnp.dot(h, W2_ref[...], preferred_element_type=jnp.float32) + b2_ref[...]
    out_ref[...] = m + _ln(o, g_ref[...], bn_ref[...])


def _full(shape):
    n = len(shape)
    return pl.BlockSpec(shape, lambda i: (0,) * n)


def _make_gather_add(E_sl, e_off, D):
    per_w = E_sl // _NW
    CH = 40
    NB = 5
    n_grp = per_w // (CH * NB)
    assert CH * NB * n_grp == per_w
    mesh = plsc.VectorSubcoreMesh(core_axis_name="c", subcore_axis_name="s",
                                  num_cores=_NC, num_subcores=_NS)

    @functools.partial(
        pl.kernel,
        out_type=jax.ShapeDtypeStruct((E_sl, D), jnp.float32),
        mesh=mesh,
        scratch_types=[
            pltpu.VMEM((per_w,), jnp.int32),
            pltpu.VMEM((per_w,), jnp.int32),
            pltpu.VMEM((NB, CH, D), jnp.float32),
            pltpu.VMEM((NB, CH, D), jnp.float32),
        ] + [pltpu.SemaphoreType.DMA] * (3 * NB),
    )
    def gather_add(ge_hbm, me_hbm, src_hbm, dst_hbm, out_hbm,
                   sidx, didx, ra, rb, *sems):
        ga, gb, ws = sems[:NB], sems[NB:2 * NB], sems[2 * NB:]
        wid = lax.axis_index("c") * _NS + lax.axis_index("s")
        base = wid * per_w
        pltpu.sync_copy(src_hbm.at[pl.ds(e_off + base, per_w)], sidx)
        pltpu.sync_copy(dst_hbm.at[pl.ds(e_off + base, per_w)], didx)

        def group(g, carry):
            k0 = g * (CH * NB)
            descs = []
            for b in range(NB):
                off = k0 + b * CH
                @pl.when(g > 0)
                def _():
                    pltpu.make_async_copy(
                        ra.at[b], out_hbm.at[pl.ds(base + off, CH)],
                        ws[b]).wait()
                da = pltpu.async_copy(
                    ge_hbm.at[sidx.at[pl.ds(off, CH)]], ra.at[b], ga[b])
                db = pltpu.async_copy(
                    me_hbm.at[didx.at[pl.ds(off, CH)]], rb.at[b], gb[b])
                descs.append((da, db))
            for b in range(NB):
                off = k0 + b * CH
                da, db = descs[b]
                da.wait()
                db.wait()

                def row(r, c2):
                    for j in range(D // 16):
                        sl = pl.ds(j * 16, 16)
                        ra[b, r, sl] = ra[b, r, sl] + rb[b, r, sl]
                    return c2

                lax.fori_loop(0, CH, row, 0)
                pltpu.async_copy(ra.at[b], out_hbm.at[pl.ds(base + off, CH)],
                                 ws[b])
            return carry

        lax.fori_loop(0, n_grp, group, 0)
        for b in range(NB):
            pltpu.make_async_copy(
                ra.at[b], out_hbm.at[pl.ds(base + b * CH, CH)], ws[b]).wait()

    return gather_add


def _make_scatter_sum(E_sl, e_off, Nm, D):
    per_w = E_sl // _NW
    CH = 40
    NB = 5
    n_grp = per_w // (CH * NB)
    assert CH * NB * n_grp == per_w
    n_blk = Nm // CH
    mesh = plsc.VectorSubcoreMesh(core_axis_name="c", subcore_axis_name="s",
                                  num_cores=_NC, num_subcores=_NS)

    @functools.partial(
        pl.kernel,
        out_type=jax.ShapeDtypeStruct((_NC * Nm, D), jnp.float32),
        mesh=mesh,
        scratch_types=[
            pltpu.VMEM((NB, CH), jnp.int32),
            pltpu.VMEM((NB, CH, D), jnp.float32),
            pltpu.VMEM_SHARED((Nm, D), jnp.float32),
        ] + [pltpu.SemaphoreType.DMA] * (3 * NB),
    )
    def scatter_sum(ef_hbm, dst_hbm, out_hbm, didx, rows, acc, *sems):
        ri, rr, ss = sems[:NB], sems[NB:2 * NB], sems[2 * NB:]
        c = lax.axis_index("c")
        s = lax.axis_index("s")
        base = (c * _NS + s) * per_w
        n_own = (n_blk - s + _NS - 1) // _NS

        def zrow(r, carry):
            for j in range(D // 16):
                rows[0, r, pl.ds(j * 16, 16)] = jnp.zeros((16,), jnp.float32)
            return carry

        lax.fori_loop(0, CH, zrow, 0)

        def zblk(m, carry):
            blk = s + m * _NS
            pltpu.sync_copy(rows.at[0], acc.at[pl.ds(blk * CH, CH)])
            return carry

        lax.fori_loop(0, n_own, zblk, 0)
        plsc.subcore_barrier()

        def group(g, carry):
            k0 = base + g * (CH * NB)
            descs = []
            for b in range(NB):
                off = k0 + b * CH
                @pl.when(g > 0)
                def _():
                    pltpu.make_async_copy(rows.at[b], acc.at[didx.at[b]],
                                          ss[b]).wait()
                di = pltpu.async_copy(dst_hbm.at[pl.ds(e_off + off, CH)],
                                      didx.at[b], ri[b])
                dr = pltpu.async_copy(ef_hbm.at[pl.ds(off, CH)],
                                      rows.at[b], rr[b])
                descs.append((di, dr))
            for b in range(NB):
                di, dr = descs[b]
                di.wait()
                dr.wait()
                pltpu.async_copy(rows.at[b], acc.at[didx.at[b]], ss[b],
                                 add=True)
            return carry

        lax.fori_loop(0, n_grp, group, 0)
        for b in range(NB):
            pltpu.make_async_copy(rows.at[b], acc.at[didx.at[b]],
                                  ss[b]).wait()
        plsc.subcore_barrier()

        def oblk(m, carry):
            blk = s + m * _NS
            pltpu.sync_copy(acc.at[pl.ds(blk * CH, CH)],
                            out_hbm.at[pl.ds(c * Nm + blk * CH, CH)])
            return carry

        lax.fori_loop(0, n_own, oblk, 0)

    return scatter_sum


def kernel(g2m_efeat, grid_nfeat, mesh_nfeat, src_idx, dst_idx,
           e_W1, e_b1, e_W2, e_b2, e_g, e_bn,
           s_W1, s_b1, s_W2, s_b2, s_g, s_bn,
           d_W1, d_b1, d_W2, d_b2, d_g, d_bn):
    E, D = g2m_efeat.shape
    Ng = grid_nfeat.shape[0]
    Nm = mesh_nfeat.shape[0]
    H = e_W1.shape[1]
    f32 = jnp.float32
    assert sum(_SLICES) == E

    W1e = e_W1[:D]
    W1g = e_W1[D:2 * D]
    W1d = e_W1[2 * D:]
    d_W1a = d_W1[:D]
    d_W1m = d_W1[D:]

    r2 = lambda v: v.reshape(1, -1)

    BG = 1000
    Ge = pl.pallas_call(
        _proj_body,
        grid=(Ng // BG,),
        in_specs=[pl.BlockSpec((BG, D), lambda i: (i, 0)), _full((D, H))],
        out_specs=pl.BlockSpec((BG, H), lambda i: (i, 0)),
        out_shape=jax.ShapeDtypeStruct((Ng, H), f32),
    )(grid_nfeat, W1g)

    BM = 1000
    Me = pl.pallas_call(
        _proj_body,
        grid=(Nm // BM,),
        in_specs=[pl.BlockSpec((BM, D), lambda i: (i, 0)), _full((D, H))],
        out_specs=pl.BlockSpec((BM, H), lambda i: (i, 0)),
        out_shape=jax.ShapeDtypeStruct((Nm, H), f32),
    )(mesh_nfeat, W1d)

    offs = [sum(_SLICES[:i]) for i in range(len(_SLICES))]
    gathered = [
        _make_gather_add(E_sl, off, H)(Ge, Me, src_idx, dst_idx)
        for E_sl, off in zip(_SLICES, offs)
    ]

    BE = 4000
    efeat = []
    for sl, (E_sl, off) in enumerate(zip(_SLICES, offs)):
        nblk = E_sl // BE
        off_blk = off // BE
        efeat.append(pl.pallas_call(
            _edge_body,
            grid=(nblk,),
            in_specs=[
                pl.BlockSpec((BE, D), lambda i, o=off_blk: (i + o, 0)),
                pl.BlockSpec((BE, H), lambda i: (i, 0)),
                _full((D, H)), _full((1, H)), _full((H, D)), _full((1, D)),
                _full((1, D)), _full((1, D)),
            ],
            out_specs=pl.BlockSpec((BE, D), lambda i: (i, 0)),
            out_shape=jax.ShapeDtypeStruct((E_sl, D), f32),
        )(g2m_efeat, gathered[sl], W1e, r2(e_b1), e_W2, r2(e_b2), r2(e_g),
          r2(e_bn)))

    partials = [
        _make_scatter_sum(E_sl, off, Nm, D)(efeat[sl], dst_idx)
        .reshape(_NC, Nm, D)
        for sl, (E_sl, off) in enumerate(zip(_SLICES, offs))
    ]

    grid_out = pl.pallas_call(
        _grid_body,
        grid=(Ng // BG,),
        in_specs=[
            pl.BlockSpec((BG, D), lambda i: (i, 0)),
            _full((D, H)), _full((1, H)), _full((H, D)), _full((1, D)),
            _full((1, D)), _full((1, D)),
        ],
        out_specs=pl.BlockSpec((BG, D), lambda i: (i, 0)),
        out_shape=jax.ShapeDtypeStruct((Ng, D), f32),
    )(grid_nfeat, s_W1, r2(s_b1), s_W2, r2(s_b2), r2(s_g), r2(s_bn))

    mesh_out = pl.pallas_call(
        _mesh_body,
        grid=(Nm // BM,),
        in_specs=(
            [pl.BlockSpec((_NC, BM, D), lambda i: (0, i, 0))
             for _ in partials] +
            [pl.BlockSpec((BM, D), lambda i: (i, 0)),
             _full((D, H)), _full((D, H)), _full((1, H)), _full((H, D)),
             _full((1, D)), _full((1, D)), _full((1, D))]
        ),
        out_specs=pl.BlockSpec((BM, D), lambda i: (i, 0)),
        out_shape=jax.ShapeDtypeStruct((Nm, D), f32),
    )(*partials, mesh_nfeat, d_W1a, d_W1m,
      r2(d_b1), d_W2, r2(d_b2), r2(d_g), r2(d_bn))

    return (grid_out, mesh_out)


# speedup vs baseline: 1.0985x; 1.0070x over previous
import functools

import jax
import jax.numpy as jnp
from jax import lax
from jax.experimental import pallas as pl
from jax.experimental.pallas import tpu as pltpu
from jax.experimental.pallas import tpu_sc as plsc

_NC, _NS = 2, 16
_NW = _NC * _NS
_SLICES = [160000, 160000]


def _ln(o, g, b, eps=1e-5):
    mu = jnp.mean(o, axis=-1, keepdims=True)
    d = o - mu
    var = jnp.mean(d * d, axis=-1, keepdims=True)
    return d * lax.rsqrt(var + eps) * g + b


def _silu(x):
    return x * jax.nn.sigmoid(x)


def _grid_body(x_ref, W1_ref, b1_ref, W2_ref, b2_ref, g_ref, bn_ref, out_ref):
    x = x_ref[...]
    h = _silu(jnp.dot(x, W1_ref[...], preferred_element_type=jnp.float32)
              + b1_ref[...])
    o = jnp.dot(h, W2_ref[...], preferred_element_type=jnp.float32) + b2_ref[...]
    out_ref[...] = x + _ln(o, g_ref[...], bn_ref[...])


def _proj_body(x_ref, W_ref, out_ref):
    out_ref[...] = jnp.dot(x_ref[...], W_ref[...],
                           preferred_element_type=jnp.float32)


def _edge_body(e_ref, gsum_ref, W1_ref, b1_ref, W2_ref, b2_ref, g_ref, bn_ref,
               out_ref):
    pre = (jnp.dot(e_ref[...], W1_ref[...], preferred_element_type=jnp.float32)
           + gsum_ref[...] + b1_ref[...])
    h = _silu(pre)
    o = jnp.dot(h, W2_ref[...], preferred_element_type=jnp.float32) + b2_ref[...]
    out_ref[...] = _ln(o, g_ref[...], bn_ref[...])


def _mesh_body(*refs):
    np_ = len(refs) - 9
    p_refs, (m_ref, Wa_ref, Wm_ref, b1_ref, W2_ref, b2_ref, g_ref, bn_ref,
             out_ref) = refs[:np_], refs[np_:]
    agg = p_refs[0][0] + p_refs[0][1]
    for p in p_refs[1:]:
        agg = agg + p[0] + p[1]
    m = m_ref[...]
    pre = (jnp.dot(agg, Wa_ref[...], preferred_element_type=jnp.float32)
           + jnp.dot(m, Wm_ref[...], preferred_element_type=jnp.float32)
           + b1_ref[...])
    h = _silu(pre)
    o = jnp.dot(h, W2_ref[...], preferred_element_type=jnp.float32) + b2_ref[...]
    out_ref[...] = m + _ln(o, g_ref[...], bn_ref[...])


def _full(shape):
    n = len(shape)
    return pl.BlockSpec(shape, lambda i: (0,) * n)


def _make_gather_add(E_sl, e_off, D):
    per_w = E_sl // _NW
    CH = 40
    NB = 5
    n_grp = per_w // (CH * NB)
    assert CH * NB * n_grp == per_w
    mesh = plsc.VectorSubcoreMesh(core_axis_name="c", subcore_axis_name="s",
                                  num_cores=_NC, num_subcores=_NS)

    @functools.partial(
        pl.kernel,
        out_type=jax.ShapeDtypeStruct((E_sl, D), jnp.float32),
        mesh=mesh,
        scratch_types=[
            pltpu.VMEM((per_w,), jnp.int32),
            pltpu.VMEM((per_w,), jnp.int32),
            pltpu.VMEM((NB, CH, D), jnp.float32),
            pltpu.VMEM((NB, CH, D), jnp.float32),
        ] + [pltpu.SemaphoreType.DMA] * (3 * NB),
    )
    def gather_add(ge_hbm, me_hbm, src_hbm, dst_hbm, out_hbm,
                   sidx, didx, ra, rb, *sems):
        ga, gb, ws = sems[:NB], sems[NB:2 * NB], sems[2 * NB:]
        wid = lax.axis_index("c") * _NS + lax.axis_index("s")
        base = wid * per_w
        pltpu.sync_copy(src_hbm.at[pl.ds(e_off + base, per_w)], sidx)
        pltpu.sync_copy(dst_hbm.at[pl.ds(e_off + base, per_w)], didx)

        def group(g, carry):
            k0 = g * (CH * NB)
            descs = []
            for b in range(NB):
                off = k0 + b * CH
                @pl.when(g > 0)
                def _():
                    pltpu.make_async_copy(
                        ra.at[b], out_hbm.at[pl.ds(base + off, CH)],
                        ws[b]).wait()
                da = pltpu.async_copy(
                    ge_hbm.at[sidx.at[pl.ds(off, CH)]], ra.at[b], ga[b])
                db = pltpu.async_copy(
                    me_hbm.at[didx.at[pl.ds(off, CH)]], rb.at[b], gb[b])
                descs.append((da, db))
            for b in range(NB):
                off = k0 + b * CH
                da, db = descs[b]
                da.wait()
                db.wait()

                def row(r, c2):
                    for j in range(D // 16):
                        sl = pl.ds(j * 16, 16)
                        ra[b, r, sl] = ra[b, r, sl] + rb[b, r, sl]
                    return c2

                lax.fori_loop(0, CH, row, 0)
                pltpu.async_copy(ra.at[b], out_hbm.at[pl.ds(base + off, CH)],
                                 ws[b])
            return carry

        lax.fori_loop(0, n_grp, group, 0)
        for b in range(NB):
            pltpu.make_async_copy(
                ra.at[b], out_hbm.at[pl.ds(base + b * CH, CH)], ws[b]).wait()

    return gather_add


def _make_scatter_sum(E_sl, e_off, Nm, D):
    per_w = E_sl // _NW
    CH = 40
    NB = 5
    n_grp = per_w // (CH * NB)
    assert CH * NB * n_grp == per_w
    n_blk = Nm // CH
    mesh = plsc.VectorSubcoreMesh(core_axis_name="c", subcore_axis_name="s",
                                  num_cores=_NC, num_subcores=_NS)

    @functools.partial(
        pl.kernel,
        out_type=jax.ShapeDtypeStruct((_NC * Nm, D), jnp.float32),
        mesh=mesh,
        scratch_types=[
            pltpu.VMEM((NB, CH), jnp.int32),
            pltpu.VMEM((NB, CH, D), jnp.float32),
            pltpu.VMEM_SHARED((Nm, D), jnp.float32),
        ] + [pltpu.SemaphoreType.DMA] * (3 * NB),
    )
    def scatter_sum(ef_hbm, dst_hbm, out_hbm, didx, rows, acc, *sems):
        ri, rr, ss = sems[:NB], sems[NB:2 * NB], sems[2 * NB:]
        c = lax.axis_index("c")
        s = lax.axis_index("s")
        base = (c * _NS + s) * per_w
        n_own = (n_blk - s + _NS - 1) // _NS

        def zrow(r, carry):
            for j in range(D // 16):
                rows[0, r, pl.ds(j * 16, 16)] = jnp.zeros((16,), jnp.float32)
            return carry

        lax.fori_loop(0, CH, zrow, 0)

        def zblk(m, carry):
            blk = s + m * _NS
            pltpu.sync_copy(rows.at[0], acc.at[pl.ds(blk * CH, CH)])
            return carry

        lax.fori_loop(0, n_own, zblk, 0)
        plsc.subcore_barrier()

        def group(g, carry):
            k0 = base + g * (CH * NB)
            descs = []
            for b in range(NB):
                off = k0 + b * CH
                @pl.when(g > 0)
                def _():
                    pltpu.make_async_copy(rows.at[b], acc.at[didx.at[b]],
                                          ss[b]).wait()
                di = pltpu.async_copy(dst_hbm.at[pl.ds(e_off + off, CH)],
                                      didx.at[b], ri[b])
                dr = pltpu.async_copy(ef_hbm.at[pl.ds(off, CH)],
                                      rows.at[b], rr[b])
                descs.append((di, dr))
            for b in range(NB):
                di, dr = descs[b]
                di.wait()
                dr.wait()
                pltpu.async_copy(rows.at[b], acc.at[didx.at[b]], ss[b],
                                 add=True)
            return carry

        lax.fori_loop(0, n_grp, group, 0)
        for b in range(NB):
            pltpu.make_async_copy(rows.at[b], acc.at[didx.at[b]],
                                  ss[b]).wait()
        plsc.subcore_barrier()

        def oblk(m, carry):
            blk = s + m * _NS
            pltpu.sync_copy(acc.at[pl.ds(blk * CH, CH)],
                            out_hbm.at[pl.ds(c * Nm + blk * CH, CH)])
            return carry

        lax.fori_loop(0, n_own, oblk, 0)

    return scatter_sum


def kernel(g2m_efeat, grid_nfeat, mesh_nfeat, src_idx, dst_idx,
           e_W1, e_b1, e_W2, e_b2, e_g, e_bn,
           s_W1, s_b1, s_W2, s_b2, s_g, s_bn,
           d_W1, d_b1, d_W2, d_b2, d_g, d_bn):
    E, D = g2m_efeat.shape
    Ng = grid_nfeat.shape[0]
    Nm = mesh_nfeat.shape[0]
    H = e_W1.shape[1]
    f32 = jnp.float32
    assert sum(_SLICES) == E

    W1e = e_W1[:D]
    W1g = e_W1[D:2 * D]
    W1d = e_W1[2 * D:]
    d_W1a = d_W1[:D]
    d_W1m = d_W1[D:]

    r2 = lambda v: v.reshape(1, -1)

    BG = 1000
    Ge = pl.pallas_call(
        _proj_body,
        grid=(Ng // BG,),
        in_specs=[pl.BlockSpec((BG, D), lambda i: (i, 0)), _full((D, H))],
        out_specs=pl.BlockSpec((BG, H), lambda i: (i, 0)),
        out_shape=jax.ShapeDtypeStruct((Ng, H), f32),
    )(grid_nfeat, W1g)

    BM = 1000
    Me = pl.pallas_call(
        _proj_body,
        grid=(Nm // BM,),
        in_specs=[pl.BlockSpec((BM, D), lambda i: (i, 0)), _full((D, H))],
        out_specs=pl.BlockSpec((BM, H), lambda i: (i, 0)),
        out_shape=jax.ShapeDtypeStruct((Nm, H), f32),
    )(mesh_nfeat, W1d)

    offs = [sum(_SLICES[:i]) for i in range(len(_SLICES))]
    gathered = [
        _make_gather_add(E_sl, off, H)(Ge, Me, src_idx, dst_idx)
        for E_sl, off in zip(_SLICES, offs)
    ]

    BE = 8000
    efeat = []
    for sl, (E_sl, off) in enumerate(zip(_SLICES, offs)):
        nblk = E_sl // BE
        off_blk = off // BE
        efeat.append(pl.pallas_call(
            _edge_body,
            grid=(nblk,),
            in_specs=[
                pl.BlockSpec((BE, D), lambda i, o=off_blk: (i + o, 0)),
                pl.BlockSpec((BE, H), lambda i: (i, 0)),
                _full((D, H)), _full((1, H)), _full((H, D)), _full((1, D)),
                _full((1, D)), _full((1, D)),
            ],
            out_specs=pl.BlockSpec((BE, D), lambda i: (i, 0)),
            out_shape=jax.ShapeDtypeStruct((E_sl, D), f32),
        )(g2m_efeat, gathered[sl], W1e, r2(e_b1), e_W2, r2(e_b2), r2(e_g),
          r2(e_bn)))

    partials = [
        _make_scatter_sum(E_sl, off, Nm, D)(efeat[sl], dst_idx)
        .reshape(_NC, Nm, D)
        for sl, (E_sl, off) in enumerate(zip(_SLICES, offs))
    ]

    grid_out = pl.pallas_call(
        _grid_body,
        grid=(Ng // BG,),
        in_specs=[
            pl.BlockSpec((BG, D), lambda i: (i, 0)),
            _full((D, H)), _full((1, H)), _full((H, D)), _full((1, D)),
            _full((1, D)), _full((1, D)),
        ],
        out_specs=pl.BlockSpec((BG, D), lambda i: (i, 0)),
        out_shape=jax.ShapeDtypeStruct((Ng, D), f32),
    )(grid_nfeat, s_W1, r2(s_b1), s_W2, r2(s_b2), r2(s_g), r2(s_bn))

    mesh_out = pl.pallas_call(
        _mesh_body,
        grid=(Nm // BM,),
        in_specs=(
            [pl.BlockSpec((_NC, BM, D), lambda i: (0, i, 0))
             for _ in partials] +
            [pl.BlockSpec((BM, D), lambda i: (i, 0)),
             _full((D, H)), _full((D, H)), _full((1, H)), _full((H, D)),
             _full((1, D)), _full((1, D)), _full((1, D))]
        ),
        out_specs=pl.BlockSpec((BM, D), lambda i: (i, 0)),
        out_shape=jax.ShapeDtypeStruct((Nm, D), f32),
    )(*partials, mesh_nfeat, d_W1a, d_W1m,
      r2(d_b1), d_W2, r2(d_b2), r2(d_g), r2(d_bn))

    return (grid_out, mesh_out)


# speedup vs baseline: 1.1335x; 1.0319x over previous
import functools

import jax
import jax.numpy as jnp
from jax import lax
from jax.experimental import pallas as pl
from jax.experimental.pallas import tpu as pltpu
from jax.experimental.pallas import tpu_sc as plsc

_NC, _NS = 2, 16
_NW = _NC * _NS
_SLICES = [160000, 160000]


def _ln(o, g, b, eps=1e-5):
    mu = jnp.mean(o, axis=-1, keepdims=True)
    d = o - mu
    var = jnp.mean(d * d, axis=-1, keepdims=True)
    return d * lax.rsqrt(var + eps) * g + b


def _silu(x):
    return x * jax.nn.sigmoid(x)


def _grid_body(x_ref, W1_ref, b1_ref, W2_ref, b2_ref, g_ref, bn_ref, out_ref):
    x = x_ref[...]
    h = _silu(jnp.dot(x, W1_ref[...], preferred_element_type=jnp.float32)
              + b1_ref[...])
    o = jnp.dot(h, W2_ref[...], preferred_element_type=jnp.float32) + b2_ref[...]
    out_ref[...] = x + _ln(o, g_ref[...], bn_ref[...])


def _proj_body(x_ref, W_ref, out_ref):
    out_ref[...] = jnp.dot(x_ref[...], W_ref[...],
                           preferred_element_type=jnp.float32)


def _edge_body(e_ref, gsum_ref, W1_ref, b1_ref, W2_ref, b2_ref, g_ref, bn_ref,
               out_ref):
    pre = (jnp.dot(e_ref[...], W1_ref[...], preferred_element_type=jnp.float32)
           + gsum_ref[...] + b1_ref[...])
    h = _silu(pre)
    o = jnp.dot(h, W2_ref[...], preferred_element_type=jnp.float32) + b2_ref[...]
    out_ref[...] = _ln(o, g_ref[...], bn_ref[...])


def _mesh_body(*refs):
    np_ = len(refs) - 9
    p_refs, (m_ref, Wa_ref, Wm_ref, b1_ref, W2_ref, b2_ref, g_ref, bn_ref,
             out_ref) = refs[:np_], refs[np_:]
    agg = p_refs[0][0] + p_refs[0][1]
    for p in p_refs[1:]:
        agg = agg + p[0] + p[1]
    m = m_ref[...]
    pre = (jnp.dot(agg, Wa_ref[...], preferred_element_type=jnp.float32)
           + jnp.dot(m, Wm_ref[...], preferred_element_type=jnp.float32)
           + b1_ref[...])
    h = _silu(pre)
    o = jnp.dot(h, W2_ref[...], preferred_element_type=jnp.float32) + b2_ref[...]
    out_ref[...] = m + _ln(o, g_ref[...], bn_ref[...])


def _full(shape):
    n = len(shape)
    return pl.BlockSpec(shape, lambda i: (0,) * n)


def _make_gather_add(E_sl, e_off, D):
    per_w = E_sl // _NW
    CH = 40
    NB = 5
    n_grp = per_w // (CH * NB)
    assert CH * NB * n_grp == per_w
    mesh = plsc.VectorSubcoreMesh(core_axis_name="c", subcore_axis_name="s",
                                  num_cores=_NC, num_subcores=_NS)

    @functools.partial(
        pl.kernel,
        out_type=jax.ShapeDtypeStruct((E_sl, D), jnp.float32),
        mesh=mesh,
        scratch_types=[
            pltpu.VMEM((per_w,), jnp.int32),
            pltpu.VMEM((per_w,), jnp.int32),
            pltpu.VMEM((NB, CH, D), jnp.float32),
            pltpu.VMEM((NB, CH, D), jnp.float32),
        ] + [pltpu.SemaphoreType.DMA] * (3 * NB),
    )
    def gather_add(ge_hbm, me_hbm, src_hbm, dst_hbm, out_hbm,
                   sidx, didx, ra, rb, *sems):
        ga, gb, ws = sems[:NB], sems[NB:2 * NB], sems[2 * NB:]
        wid = lax.axis_index("c") * _NS + lax.axis_index("s")
        base = wid * per_w
        pltpu.sync_copy(src_hbm.at[pl.ds(e_off + base, per_w)], sidx)
        pltpu.sync_copy(dst_hbm.at[pl.ds(e_off + base, per_w)], didx)

        def group(g, carry):
            k0 = g * (CH * NB)
            descs = []
            for b in range(NB):
                off = k0 + b * CH
                @pl.when(g > 0)
                def _():
                    pltpu.make_async_copy(
                        ra.at[b], out_hbm.at[pl.ds(base + off, CH)],
                        ws[b]).wait()
                da = pltpu.async_copy(
                    ge_hbm.at[sidx.at[pl.ds(off, CH)]], ra.at[b], ga[b])
                db = pltpu.async_copy(
                    me_hbm.at[didx.at[pl.ds(off, CH)]], rb.at[b], gb[b])
                descs.append((da, db))
            for b in range(NB):
                off = k0 + b * CH
                da, db = descs[b]
                da.wait()
                db.wait()

                def row(r, c2):
                    for j in range(D // 16):
                        sl = pl.ds(j * 16, 16)
                        ra[b, r, sl] = ra[b, r, sl] + rb[b, r, sl]
                    return c2

                lax.fori_loop(0, CH, row, 0)
                pltpu.async_copy(ra.at[b], out_hbm.at[pl.ds(base + off, CH)],
                                 ws[b])
            return carry

        lax.fori_loop(0, n_grp, group, 0)
        for b in range(NB):
            pltpu.make_async_copy(
                ra.at[b], out_hbm.at[pl.ds(base + b * CH, CH)], ws[b]).wait()

    return gather_add


def _make_scatter_sum(E_sl, e_off, Nm, D):
    per_w = E_sl // _NW
    CH = 40
    NB = 5
    n_grp = per_w // (CH * NB)
    assert CH * NB * n_grp == per_w
    n_blk = Nm // CH
    mesh = plsc.VectorSubcoreMesh(core_axis_name="c", subcore_axis_name="s",
                                  num_cores=_NC, num_subcores=_NS)

    @functools.partial(
        pl.kernel,
        out_type=jax.ShapeDtypeStruct((_NC * Nm, D), jnp.float32),
        mesh=mesh,
        scratch_types=[
            pltpu.VMEM((NB, CH), jnp.int32),
            pltpu.VMEM((NB, CH, D), jnp.float32),
            pltpu.VMEM_SHARED((Nm, D), jnp.float32),
        ] + [pltpu.SemaphoreType.DMA] * (3 * NB),
    )
    def scatter_sum(ef_hbm, dst_hbm, out_hbm, didx, rows, acc, *sems):
        ri, rr, ss = sems[:NB], sems[NB:2 * NB], sems[2 * NB:]
        c = lax.axis_index("c")
        s = lax.axis_index("s")
        base = (c * _NS + s) * per_w
        n_own = (n_blk - s + _NS - 1) // _NS

        def zrow(r, carry):
            for j in range(D // 16):
                rows[0, r, pl.ds(j * 16, 16)] = jnp.zeros((16,), jnp.float32)
            return carry

        lax.fori_loop(0, CH, zrow, 0)

        def zblk(m, carry):
            blk = s + m * _NS
            pltpu.sync_copy(rows.at[0], acc.at[pl.ds(blk * CH, CH)])
            return carry

        lax.fori_loop(0, n_own, zblk, 0)
        plsc.subcore_barrier()

        def group(g, carry):
            k0 = base + g * (CH * NB)
            descs = []
            for b in range(NB):
                off = k0 + b * CH
                @pl.when(g > 0)
                def _():
                    pltpu.make_async_copy(rows.at[b], acc.at[didx.at[b]],
                                          ss[b]).wait()
                di = pltpu.async_copy(dst_hbm.at[pl.ds(e_off + off, CH)],
                                      didx.at[b], ri[b])
                dr = pltpu.async_copy(ef_hbm.at[pl.ds(off, CH)],
                                      rows.at[b], rr[b])
                descs.append((di, dr))
            for b in range(NB):
                di, dr = descs[b]
                di.wait()
                dr.wait()
                pltpu.async_copy(rows.at[b], acc.at[didx.at[b]], ss[b],
                                 add=True)
            return carry

        lax.fori_loop(0, n_grp, group, 0)
        for b in range(NB):
            pltpu.make_async_copy(rows.at[b], acc.at[didx.at[b]],
                                  ss[b]).wait()
        plsc.subcore_barrier()

        def oblk(m, carry):
            blk = s + m * _NS
            pltpu.sync_copy(acc.at[pl.ds(blk * CH, CH)],
                            out_hbm.at[pl.ds(c * Nm + blk * CH, CH)])
            return carry

        lax.fori_loop(0, n_own, oblk, 0)

    return scatter_sum


def kernel(g2m_efeat, grid_nfeat, mesh_nfeat, src_idx, dst_idx,
           e_W1, e_b1, e_W2, e_b2, e_g, e_bn,
           s_W1, s_b1, s_W2, s_b2, s_g, s_bn,
           d_W1, d_b1, d_W2, d_b2, d_g, d_bn):
    E, D = g2m_efeat.shape
    Ng = grid_nfeat.shape[0]
    Nm = mesh_nfeat.shape[0]
    H = e_W1.shape[1]
    f32 = jnp.float32
    assert sum(_SLICES) == E

    W1e = e_W1[:D]
    W1g = e_W1[D:2 * D]
    W1d = e_W1[2 * D:]
    d_W1a = d_W1[:D]
    d_W1m = d_W1[D:]

    r2 = lambda v: v.reshape(1, -1)

    BG = 2000
    Ge = pl.pallas_call(
        _proj_body,
        grid=(Ng // BG,),
        in_specs=[pl.BlockSpec((BG, D), lambda i: (i, 0)), _full((D, H))],
        out_specs=pl.BlockSpec((BG, H), lambda i: (i, 0)),
        out_shape=jax.ShapeDtypeStruct((Ng, H), f32),
    )(grid_nfeat, W1g)

    BM = 2000
    Me = pl.pallas_call(
        _proj_body,
        grid=(Nm // BM,),
        in_specs=[pl.BlockSpec((BM, D), lambda i: (i, 0)), _full((D, H))],
        out_specs=pl.BlockSpec((BM, H), lambda i: (i, 0)),
        out_shape=jax.ShapeDtypeStruct((Nm, H), f32),
    )(mesh_nfeat, W1d)

    offs = [sum(_SLICES[:i]) for i in range(len(_SLICES))]
    gathered = [
        _make_gather_add(E_sl, off, H)(Ge, Me, src_idx, dst_idx)
        for E_sl, off in zip(_SLICES, offs)
    ]

    BE = 8000
    efeat = []
    for sl, (E_sl, off) in enumerate(zip(_SLICES, offs)):
        nblk = E_sl // BE
        off_blk = off // BE
        efeat.append(pl.pallas_call(
            _edge_body,
            grid=(nblk,),
            in_specs=[
                pl.BlockSpec((BE, D), lambda i, o=off_blk: (i + o, 0)),
                pl.BlockSpec((BE, H), lambda i: (i, 0)),
                _full((D, H)), _full((1, H)), _full((H, D)), _full((1, D)),
                _full((1, D)), _full((1, D)),
            ],
            out_specs=pl.BlockSpec((BE, D), lambda i: (i, 0)),
            out_shape=jax.ShapeDtypeStruct((E_sl, D), f32),
        )(g2m_efeat, gathered[sl], W1e, r2(e_b1), e_W2, r2(e_b2), r2(e_g),
          r2(e_bn)))

    partials = [
        _make_scatter_sum(E_sl, off, Nm, D)(efeat[sl], dst_idx)
        .reshape(_NC, Nm, D)
        for sl, (E_sl, off) in enumerate(zip(_SLICES, offs))
    ]

    grid_out = pl.pallas_call(
        _grid_body,
        grid=(Ng // BG,),
        in_specs=[
            pl.BlockSpec((BG, D), lambda i: (i, 0)),
            _full((D, H)), _full((1, H)), _full((H, D)), _full((1, D)),
            _full((1, D)), _full((1, D)),
        ],
        out_specs=pl.BlockSpec((BG, D), lambda i: (i, 0)),
        out_shape=jax.ShapeDtypeStruct((Ng, D), f32),
    )(grid_nfeat, s_W1, r2(s_b1), s_W2, r2(s_b2), r2(s_g), r2(s_bn))

    mesh_out = pl.pallas_call(
        _mesh_body,
        grid=(Nm // BM,),
        in_specs=(
            [pl.BlockSpec((_NC, BM, D), lambda i: (0, i, 0))
             for _ in partials] +
            [pl.BlockSpec((BM, D), lambda i: (i, 0)),
             _full((D, H)), _full((D, H)), _full((1, H)), _full((H, D)),
             _full((1, D)), _full((1, D)), _full((1, D))]
        ),
        out_specs=pl.BlockSpec((BM, D), lambda i: (i, 0)),
        out_shape=jax.ShapeDtypeStruct((Nm, D), f32),
    )(*partials, mesh_nfeat, d_W1a, d_W1m,
      r2(d_b1), d_W2, r2(d_b2), r2(d_g), r2(d_bn))

    return (grid_out, mesh_out)
